# baseline (speedup 1.0000x reference)
import jax
import jax.numpy as jnp
from jax.experimental import pallas as pl
from jax.scipy.special import digamma

_K = 5
_N = 4096
_TILE = 256
_GRID = _N // _TILE


def _dot(a, b, dims):
    return jax.lax.dot_general(
        a, b, (dims, ((), ())),
        precision=jax.lax.Precision.DEFAULT,
        preferred_element_type=jnp.float32,
    )


def _counts_kernel(ax_ref, bx_ref, ay_ref, by_ref, bxi_ref, byi_ref, out_ref):
    ax = ax_ref[...]
    bx = bx_ref[...]
    ay = ay_ref[...]
    by = by_ref[...]
    bxi = bxi_ref[...]
    byi = byi_ref[...]

    d2x = _dot(bxi, ax, ((1,), (1,)))
    d2y = _dot(byi, ay, ((1,), (1,)))

    db = jax.lax.bitcast_convert_type(
        jnp.maximum(jnp.maximum(d2x, d2y), 0.0), jnp.int32)
    sign = jnp.int32(-2**31)
    b = jnp.min(db, axis=1, keepdims=True)
    for _ in range(_K):
        c = (b + 1) ^ sign
        b = c + jnp.min(db - c, axis=1, keepdims=True)

    onehot = (db == b).astype(jnp.float32)
    bxa = _dot(onehot, bx, ((1,), (0,)))
    bya = _dot(onehot, by, ((1,), (0,)))

    dax2 = _dot(bxa, ax, ((1,), (1,)))
    day2 = _dot(bya, ay, ((1,), (1,)))

    nx = jnp.sum((d2x <= dax2).astype(jnp.float32), axis=1)
    ny = jnp.sum((d2y <= day2).astype(jnp.float32), axis=1)
    out_ref[0, :] = nx
    out_ref[1, :] = ny


@jax.jit
def _counts(ax, bx, ay, by):
    return pl.pallas_call(
        _counts_kernel,
        grid=(_GRID,),
        in_specs=[
            pl.BlockSpec((_N, 130), lambda i: (0, 0)),
            pl.BlockSpec((_N, 130), lambda i: (0, 0)),
            pl.BlockSpec((_N, 18), lambda i: (0, 0)),
            pl.BlockSpec((_N, 18), lambda i: (0, 0)),
            pl.BlockSpec((_TILE, 130), lambda i: (i, 0)),
            pl.BlockSpec((_TILE, 18), lambda i: (i, 0)),
        ],
        out_specs=pl.BlockSpec((2, _TILE), lambda i: (0, i)),
        out_shape=jax.ShapeDtypeStruct((2, _N), jnp.float32),
    )(ax, bx, ay, by, bx, by)


def _pack(a):
    sq = jnp.sum(a * a, axis=1, keepdims=True)
    ones = jnp.ones_like(sq)
    return (jnp.concatenate([a, sq, ones], axis=1),
            jnp.concatenate([-2.0 * a, ones, sq], axis=1))


def kernel(X, y):
    X = X.astype(jnp.float32)
    y = y.astype(jnp.float32)
    ax, bx = _pack(X)
    ay, by = _pack(y)
    counts = _counts(ax, bx, ay, by)
    nx, ny = counts[0], counts[1]
    k_digamma = digamma(jnp.float32(_K)) - 1.0 / _K
    n_digamma = digamma(jnp.float32(_N))
    n_avg_digamma = jnp.mean(digamma(nx + 1.0) + digamma(ny + 1.0))
    mi = n_digamma + k_digamma - n_avg_digamma
    mi = mi / jnp.log(jnp.float32(2.0))
    return jax.nn.relu(mi)


# speedup vs baseline: 1.1833x; 1.1833x over previous
import jax
import jax.numpy as jnp
from jax.experimental import pallas as pl
from jax.scipy.special import digamma

_K = 5
_N = 4096
_TILE = 512
_SUB = 256
_GRID = _N // _TILE


def _dot(a, b, dims):
    return jax.lax.dot_general(
        a, b, (dims, ((), ())),
        precision=jax.lax.Precision.DEFAULT,
        preferred_element_type=jnp.float32,
    )


def _stage_d(ax, ay, bxi, byi):
    d2x = _dot(bxi, ax, ((1,), (1,)))
    d2y = _dot(byi, ay, ((1,), (1,)))
    return d2x, d2y, jnp.maximum(d2x, d2y)


def _scan(d):
    inf = jnp.float32(jnp.inf)
    m = jnp.min(d, axis=1, keepdims=True)
    for _ in range(_K):
        m = jnp.min(jnp.where(d > m, d, inf), axis=1, keepdims=True)
    return m


def _finish(ax, bx, ay, by, d2x, d2y, d, m):
    onehot = (d == m).astype(jnp.float32)
    bxa = _dot(onehot, bx, ((1,), (0,)))
    bya = _dot(onehot, by, ((1,), (0,)))

    dax2 = _dot(bxa, ax, ((1,), (1,)))
    day2 = _dot(bya, ay, ((1,), (1,)))

    nx = jnp.sum((d2x <= dax2).astype(jnp.float32), axis=1)
    ny = jnp.sum((d2y <= day2).astype(jnp.float32), axis=1)
    return nx, ny


def _counts_kernel(ax_ref, bx_ref, ay_ref, by_ref, bxi_ref, byi_ref, out_ref):
    ax = ax_ref[...]
    bx = bx_ref[...]
    ay = ay_ref[...]
    by = by_ref[...]

    sl0 = slice(0, _SUB)
    sl1 = slice(_SUB, 2 * _SUB)
    s0 = _stage_d(ax, ay, bxi_ref[sl0, :], byi_ref[sl0, :])
    s1 = _stage_d(ax, ay, bxi_ref[sl1, :], byi_ref[sl1, :])
    inf = jnp.float32(jnp.inf)
    d0, d1 = s0[2], s1[2]
    m0 = jnp.min(d0, axis=1, keepdims=True)
    m1 = jnp.min(d1, axis=1, keepdims=True)
    for _ in range(_K):
        m0 = jnp.min(jnp.where(d0 > m0, d0, inf), axis=1, keepdims=True)
        m1 = jnp.min(jnp.where(d1 > m1, d1, inf), axis=1, keepdims=True)
    nx0, ny0 = _finish(ax, bx, ay, by, *s0, m0)
    nx1, ny1 = _finish(ax, bx, ay, by, *s1, m1)
    out_ref[0, sl0] = nx0
    out_ref[1, sl0] = ny0
    out_ref[0, sl1] = nx1
    out_ref[1, sl1] = ny1


@jax.jit
def _counts(ax, bx, ay, by):
    return pl.pallas_call(
        _counts_kernel,
        grid=(_GRID,),
        in_specs=[
            pl.BlockSpec((_N, 130), lambda i: (0, 0)),
            pl.BlockSpec((_N, 130), lambda i: (0, 0)),
            pl.BlockSpec((_N, 18), lambda i: (0, 0)),
            pl.BlockSpec((_N, 18), lambda i: (0, 0)),
            pl.BlockSpec((_TILE, 130), lambda i: (i, 0)),
            pl.BlockSpec((_TILE, 18), lambda i: (i, 0)),
        ],
        out_specs=pl.BlockSpec((2, _TILE), lambda i: (0, i)),
        out_shape=jax.ShapeDtypeStruct((2, _N), jnp.float32),
    )(ax, bx, ay, by, bx, by)


def _pack(a):
    sq = jnp.sum(a * a, axis=1, keepdims=True)
    ones = jnp.ones_like(sq)
    return (jnp.concatenate([a, sq, ones], axis=1),
            jnp.concatenate([-2.0 * a, ones, sq], axis=1))


def kernel(X, y):
    X = X.astype(jnp.float32)
    y = y.astype(jnp.float32)
    ax, bx = _pack(X)
    ay, by = _pack(y)
    counts = _counts(ax, bx, ay, by)
    nx, ny = counts[0], counts[1]
    k_digamma = digamma(jnp.float32(_K)) - 1.0 / _K
    n_digamma = digamma(jnp.float32(_N))
    n_avg_digamma = jnp.mean(digamma(nx + 1.0) + digamma(ny + 1.0))
    mi = n_digamma + k_digamma - n_avg_digamma
    mi = mi / jnp.log(jnp.float32(2.0))
    return jax.nn.relu(mi)
